# Initial kernel scaffold; baseline (speedup 1.0000x reference)
import jax
import jax.numpy as jnp
from jax.experimental import pallas as pl


def kernel(x, edge_index, W1, a_s1, a_d1, b1, W2, a_s2, a_d2, b2, Wout, bout):
    raise NotImplementedError("write your pallas kernel here")


# speedup vs baseline: 18.8474x; 18.8474x over previous
import functools

import jax
import jax.numpy as jnp
from jax import lax
from jax.experimental import pallas as pl
from jax.experimental.pallas import tpu as pltpu
from jax.experimental.pallas import tpu_sc as plsc

N = 10000
E = 320000
NC = 2
NS = 16
L = 16
HB = 128
EA = E // (NC * NS)


def _tc_embed(x, W, att_s, att_d):
    n, h_true = x.shape[0], W.shape[1]

    def body(x_ref, w_ref, avs_ref, avd_ref, h_ref, as_ref, ad_ref):
        h = jnp.dot(x_ref[...], w_ref[...], preferred_element_type=jnp.float32)
        as_ref[...] = jnp.sum(h * avs_ref[...][None, :], axis=1)
        ad_ref[...] = jnp.sum(h * avd_ref[...][None, :], axis=1)
        ones = jnp.ones((n, 1), jnp.float32)
        pad = jnp.zeros((n, 2 * HB - h_true - 1), jnp.float32)
        hp = jnp.concatenate([h, ones, pad], axis=1)
        h_ref[0] = hp[:, :HB]
        h_ref[1] = hp[:, HB:]

    return pl.pallas_call(
        body,
        out_shape=(
            jax.ShapeDtypeStruct((2, n, HB), jnp.float32),
            jax.ShapeDtypeStruct((n,), jnp.float32),
            jax.ShapeDtypeStruct((n,), jnp.float32),
        ),
    )(x, W, att_s, att_d)


def _tc_norm_embed(S3, W, b, att_s, att_d, h_true_in):
    n = S3.shape[1]
    h_true = W.shape[1]

    def body(s_ref, w_ref, b_ref, avs_ref, avd_ref, h_ref, as_ref, ad_ref):
        sfull = jnp.concatenate([s_ref[0], s_ref[1]], axis=1)
        S = sfull[:, :h_true_in]
        D = sfull[:, h_true_in]
        xin = jax.nn.relu(S / (D[:, None] + 1e-16) + b_ref[...][None, :])
        h = jnp.dot(xin, w_ref[...], preferred_element_type=jnp.float32)
        as_ref[...] = jnp.sum(h * avs_ref[...][None, :], axis=1)
        ad_ref[...] = jnp.sum(h * avd_ref[...][None, :], axis=1)
        ones = jnp.ones((n, 1), jnp.float32)
        pad = jnp.zeros((n, HB - h_true - 1), jnp.float32)
        h_ref[...] = jnp.concatenate([h, ones, pad], axis=1)

    return pl.pallas_call(
        body,
        out_shape=(
            jax.ShapeDtypeStruct((n, HB), jnp.float32),
            jax.ShapeDtypeStruct((n,), jnp.float32),
            jax.ShapeDtypeStruct((n,), jnp.float32),
        ),
    )(S3, W, b, att_s, att_d)


def _tc_final(S3, W, b, bout, h_true_in):
    n = S3.shape[1]

    def body(s_ref, w_ref, b_ref, bo_ref, o_ref):
        sfull = s_ref[0] + s_ref[1]
        S = sfull[:, :h_true_in]
        D = sfull[:, h_true_in]
        xin = jax.nn.relu(S / (D[:, None] + 1e-16) + b_ref[...][None, :])
        o_ref[...] = (
            jnp.dot(xin, w_ref[...], preferred_element_type=jnp.float32)
            + bo_ref[...][None, :]
        )

    return pl.pallas_call(
        body,
        out_shape=jax.ShapeDtypeStruct((n, W.shape[1]), jnp.float32),
    )(S3, W, b, bout)


def _sc_edge_w(a_s, a_d, srcA, dstA):
    mesh = plsc.VectorSubcoreMesh(core_axis_name="c", subcore_axis_name="s")

    @functools.partial(
        pl.kernel,
        mesh=mesh,
        compiler_params=pltpu.CompilerParams(needs_layout_passes=False),
        out_type=jax.ShapeDtypeStruct((NC * NS, EA), jnp.float32),
        scratch_types=[
            pltpu.VMEM((N,), jnp.float32),
            pltpu.VMEM((N,), jnp.float32),
            pltpu.VMEM((EA,), jnp.int32),
            pltpu.VMEM((EA,), jnp.int32),
            pltpu.VMEM((EA,), jnp.float32),
        ],
    )
    def k(as_hbm, ad_hbm, src_hbm, dst_hbm, w_hbm, as_v, ad_v, src_v, dst_v, w_v):
        wid = lax.axis_index("c") * NS + lax.axis_index("s")
        pltpu.sync_copy(as_hbm, as_v)
        pltpu.sync_copy(ad_hbm, ad_v)
        pltpu.sync_copy(src_hbm.at[wid], src_v)
        pltpu.sync_copy(dst_hbm.at[wid], dst_v)

        @pl.loop(0, EA // L)
        def _(i):
            sl = pl.ds(i * L, L)
            e = plsc.load_gather(as_v, [src_v[sl]]) + plsc.load_gather(
                ad_v, [dst_v[sl]]
            )
            e = jnp.where(e > 0.0, e, 0.2 * e)
            w_v[sl] = jnp.exp(e)

        pltpu.sync_copy(w_v, w_hbm.at[wid])

    return k(a_s, a_d, srcA, dstA)


def _sc_scatter(h2d, edata, chunk, nch):
    mesh = plsc.VectorSubcoreMesh(core_axis_name="c", subcore_axis_name="s")
    jcount = HB // L

    @functools.partial(
        pl.kernel,
        mesh=mesh,
        compiler_params=pltpu.CompilerParams(
            needs_layout_passes=False, use_tc_tiling_on_sc=False
        ),
        out_type=jax.ShapeDtypeStruct((2 * N, HB), jnp.float32),
        scratch_types=[
            pltpu.VMEM_SHARED((N, HB), jnp.float32),
            pltpu.VMEM((2, 3, chunk), jnp.int32),
            pltpu.VMEM((chunk, HB), jnp.float32),
            pltpu.VMEM((chunk, HB), jnp.float32),
            pltpu.SemaphoreType.DMA,
            pltpu.SemaphoreType.DMA,
            pltpu.SemaphoreType.DMA,
            pltpu.SemaphoreType.DMA,
        ],
    )
    def k(h_hbm, e_hbm, s_hbm, acc_sh, eb_v, rv0, rv1, g0, g1, e0, e1):
        c = lax.axis_index("c")
        s = lax.axis_index("s")
        rvs = (rv0, rv1)
        gsem = (g0, g1)
        esem = (e0, e1)

        @pl.loop(0, 16)
        def _(r):
            for j in range(jcount):
                rv0[r, pl.ds(j * L, L)] = jnp.zeros((L,), jnp.float32)

        base = s * 624
        nzchunks = jnp.where(s == NS - 1, 40, 39)

        @pl.loop(0, nzchunks)
        def _(i):
            pltpu.sync_copy(
                rv0.at[pl.ds(0, 16)], acc_sh.at[pl.ds(base + i * 16, 16)]
            )

        plsc.subcore_barrier()

        pltpu.async_copy(e_hbm.at[c, s, 0], eb_v.at[0], e0)
        pltpu.async_copy(e_hbm.at[c, s, 1], eb_v.at[1], e1)
        pltpu.make_async_copy(e_hbm.at[c, s, 0], eb_v.at[0], e0).wait()
        pltpu.async_copy(h_hbm.at[eb_v.at[0, 0]], rv0, g0)

        @pl.loop(0, nch // 2)
        def _(g):
            for b in range(2):
                nb = 1 - b
                kk = g * 2 + b
                rv = rvs[b]
                pltpu.make_async_copy(h_hbm.at[eb_v.at[b, 0]], rv, gsem[b]).wait()

                @pl.loop(0, chunk)
                def _(r):
                    wvec = plsc.bitcast(
                        plsc.load_gather(
                            eb_v.at[b, 2], [jnp.broadcast_to(r, (L,))]
                        ),
                        jnp.float32,
                    )
                    for j in range(jcount):
                        sl = pl.ds(j * L, L)
                        rv[r, sl] = rv[r, sl] * wvec

                pltpu.sync_copy(rv, acc_sh.at[eb_v.at[b, 1]], add=True)

                @pl.when(kk + 2 < nch)
                def _():
                    pltpu.async_copy(e_hbm.at[c, s, kk + 2], eb_v.at[b], esem[b])

                @pl.when(kk + 1 < nch)
                def _():
                    pltpu.make_async_copy(
                        e_hbm.at[c, s, kk + 1], eb_v.at[nb], esem[nb]
                    ).wait()
                    pltpu.async_copy(h_hbm.at[eb_v.at[nb, 0]], rvs[nb], gsem[nb])

        plsc.subcore_barrier()

        @pl.loop(0, nzchunks)
        def _(i):
            pltpu.sync_copy(
                acc_sh.at[pl.ds(base + i * 16, 16)],
                s_hbm.at[pl.ds(c * N + base + i * 16, 16)],
            )

    return k(h2d, edata)


def kernel(x, edge_index, W1, a_s1, a_d1, b1, W2, a_s2, a_d2, b2, Wout, bout):
    src = edge_index[0]
    dst = edge_index[1]
    srcA = src.reshape(NC * NS, EA)
    dstA = dst.reshape(NC * NS, EA)

    c1, n1 = 80, 250
    srcp1 = jnp.stack([src, src + N]).reshape(2, NS, n1, c1)
    dstp1 = jnp.stack([dst, dst]).reshape(2, NS, n1, c1)

    c2, n2 = 100, 100
    srcp2 = src.reshape(2, NS, n2, c2)
    dstp2 = dst.reshape(2, NS, n2, c2)

    h3, as1v, ad1v = _tc_embed(x, W1, a_s1, a_d1)
    w1 = _sc_edge_w(as1v, ad1v, srcA, dstA)
    w1b = lax.bitcast_convert_type(w1.reshape(E), jnp.int32)
    wp1 = jnp.stack([w1b, w1b]).reshape(2, NS, n1, 1, c1)
    edata1 = jnp.concatenate(
        [srcp1.reshape(2, NS, n1, 1, c1), dstp1.reshape(2, NS, n1, 1, c1), wp1],
        axis=3,
    )
    S1 = _sc_scatter(h3.reshape(2 * N, HB), edata1, c1, n1)

    h2, as2v, ad2v = _tc_norm_embed(S1.reshape(2, N, HB), W2, b1,
                                    a_s2, a_d2, 200)
    w2 = _sc_edge_w(as2v, ad2v, srcA, dstA)
    wp2 = lax.bitcast_convert_type(w2, jnp.int32).reshape(2, NS, n2, 1, c2)
    edata2 = jnp.concatenate(
        [srcp2.reshape(2, NS, n2, 1, c2), dstp2.reshape(2, NS, n2, 1, c2), wp2],
        axis=3,
    )
    S2 = _sc_scatter(h2, edata2, c2, n2)

    return _tc_final(S2.reshape(2, N, HB), Wout, b2, bout, 100)


# speedup vs baseline: 23.2339x; 1.2327x over previous
import functools

import jax
import jax.numpy as jnp
from jax import lax
from jax.experimental import pallas as pl
from jax.experimental.pallas import tpu as pltpu
from jax.experimental.pallas import tpu_sc as plsc

N = 10000
E = 320000
NC = 2
NS = 16
L = 16
HB = 128
EA = E // (NC * NS)


def _tc_embed(x, W, att_s, att_d):
    n, h_true = x.shape[0], W.shape[1]

    def body(x_ref, w_ref, avs_ref, avd_ref, h_ref, as_ref, ad_ref):
        h = jnp.dot(x_ref[...], w_ref[...], preferred_element_type=jnp.float32)
        as_ref[...] = jnp.sum(h * avs_ref[...][None, :], axis=1)
        ad_ref[...] = jnp.sum(h * avd_ref[...][None, :], axis=1)
        ones = jnp.ones((n, 1), jnp.float32)
        pad = jnp.zeros((n, 2 * HB - h_true - 1), jnp.float32)
        hp = jnp.concatenate([h, ones, pad], axis=1)
        h_ref[0] = hp[:, :HB]
        h_ref[1] = hp[:, HB:]

    return pl.pallas_call(
        body,
        out_shape=(
            jax.ShapeDtypeStruct((2, n, HB), jnp.float32),
            jax.ShapeDtypeStruct((n,), jnp.float32),
            jax.ShapeDtypeStruct((n,), jnp.float32),
        ),
    )(x, W, att_s, att_d)


def _tc_norm_embed(S3, W, b, att_s, att_d, h_true_in):
    n = S3.shape[1]
    h_true = W.shape[1]

    def body(s_ref, w_ref, b_ref, avs_ref, avd_ref, h_ref, as_ref, ad_ref):
        sfull = jnp.concatenate([s_ref[0], s_ref[1]], axis=1)
        S = sfull[:, :h_true_in]
        D = sfull[:, h_true_in]
        xin = jax.nn.relu(S / (D[:, None] + 1e-16) + b_ref[...][None, :])
        h = jnp.dot(xin, w_ref[...], preferred_element_type=jnp.float32)
        as_ref[...] = jnp.sum(h * avs_ref[...][None, :], axis=1)
        ad_ref[...] = jnp.sum(h * avd_ref[...][None, :], axis=1)
        ones = jnp.ones((n, 1), jnp.float32)
        pad = jnp.zeros((n, HB - h_true - 1), jnp.float32)
        h_ref[...] = jnp.concatenate([h, ones, pad], axis=1)

    return pl.pallas_call(
        body,
        out_shape=(
            jax.ShapeDtypeStruct((n, HB), jnp.float32),
            jax.ShapeDtypeStruct((n,), jnp.float32),
            jax.ShapeDtypeStruct((n,), jnp.float32),
        ),
    )(S3, W, b, att_s, att_d)


def _tc_final(S3, W, b, bout, h_true_in):
    n = S3.shape[1]

    def body(s_ref, w_ref, b_ref, bo_ref, o_ref):
        sfull = s_ref[0] + s_ref[1]
        S = sfull[:, :h_true_in]
        D = sfull[:, h_true_in]
        xin = jax.nn.relu(S / (D[:, None] + 1e-16) + b_ref[...][None, :])
        o_ref[...] = (
            jnp.dot(xin, w_ref[...], preferred_element_type=jnp.float32)
            + bo_ref[...][None, :]
        )

    return pl.pallas_call(
        body,
        out_shape=jax.ShapeDtypeStruct((n, W.shape[1]), jnp.float32),
    )(S3, W, b, bout)


def _sc_edge_w(a_s, a_d, srcA, dstA):
    mesh = plsc.VectorSubcoreMesh(core_axis_name="c", subcore_axis_name="s")

    @functools.partial(
        pl.kernel,
        mesh=mesh,
        compiler_params=pltpu.CompilerParams(needs_layout_passes=False),
        out_type=jax.ShapeDtypeStruct((NC * NS, EA), jnp.float32),
        scratch_types=[
            pltpu.VMEM((N,), jnp.float32),
            pltpu.VMEM((N,), jnp.float32),
            pltpu.VMEM((EA,), jnp.int32),
            pltpu.VMEM((EA,), jnp.int32),
            pltpu.VMEM((EA,), jnp.float32),
        ],
    )
    def k(as_hbm, ad_hbm, src_hbm, dst_hbm, w_hbm, as_v, ad_v, src_v, dst_v, w_v):
        wid = lax.axis_index("c") * NS + lax.axis_index("s")
        pltpu.sync_copy(as_hbm, as_v)
        pltpu.sync_copy(ad_hbm, ad_v)
        pltpu.sync_copy(src_hbm.at[wid], src_v)
        pltpu.sync_copy(dst_hbm.at[wid], dst_v)

        @pl.loop(0, EA // L)
        def _(i):
            sl = pl.ds(i * L, L)
            e = plsc.load_gather(as_v, [src_v[sl]]) + plsc.load_gather(
                ad_v, [dst_v[sl]]
            )
            e = jnp.where(e > 0.0, e, 0.2 * e)
            w_v[sl] = jnp.exp(e)

        pltpu.sync_copy(w_v, w_hbm.at[wid])

    return k(a_s, a_d, srcA, dstA)


def _sc_scatter(h2d, edata, chunk, nch):
    mesh = plsc.VectorSubcoreMesh(core_axis_name="c", subcore_axis_name="s")
    jcount = HB // L

    @functools.partial(
        pl.kernel,
        mesh=mesh,
        compiler_params=pltpu.CompilerParams(
            needs_layout_passes=False, use_tc_tiling_on_sc=False
        ),
        out_type=jax.ShapeDtypeStruct((2 * N, HB), jnp.float32),
        scratch_types=[
            pltpu.VMEM_SHARED((N, HB), jnp.float32),
            pltpu.VMEM((4, 3, chunk), jnp.int32),
            pltpu.VMEM((chunk, HB), jnp.float32),
            pltpu.VMEM((chunk, HB), jnp.float32),
            pltpu.SemaphoreType.DMA,
            pltpu.SemaphoreType.DMA,
            pltpu.SemaphoreType.DMA,
            pltpu.SemaphoreType.DMA,
            pltpu.SemaphoreType.DMA,
            pltpu.SemaphoreType.DMA,
            pltpu.SemaphoreType.DMA,
            pltpu.SemaphoreType.DMA,
        ],
    )
    def k(h_hbm, e_hbm, s_hbm, acc_sh, eb_v, rv0, rv1,
          g0, g1, s0, s1, e0, e1, e2, e3):
        c = lax.axis_index("c")
        s = lax.axis_index("s")
        rvs = (rv0, rv1)
        gsem = (g0, g1)
        ssem = (s0, s1)
        esem = (e0, e1, e2, e3)

        @pl.loop(0, 16)
        def _(r):
            for j in range(jcount):
                rv0[r, pl.ds(j * L, L)] = jnp.zeros((L,), jnp.float32)

        base = s * 624
        nzchunks = jnp.where(s == NS - 1, 40, 39)

        @pl.loop(0, nzchunks)
        def _(i):
            pltpu.sync_copy(
                rv0.at[pl.ds(0, 16)], acc_sh.at[pl.ds(base + i * 16, 16)]
            )

        plsc.subcore_barrier()

        for t in range(3):
            pltpu.async_copy(e_hbm.at[c, s, t], eb_v.at[t], esem[t])
        pltpu.make_async_copy(e_hbm.at[c, s, 0], eb_v.at[0], esem[0]).wait()
        pltpu.async_copy(h_hbm.at[eb_v.at[0, 0]], rv0, g0)

        @pl.loop(0, nch // 4)
        def _(g):
            for q in range(4):
                b = q % 2
                nb = 1 - b
                kk = g * 4 + q
                rv = rvs[b]
                pltpu.make_async_copy(h_hbm.at[eb_v.at[q, 0]], rv, gsem[b]).wait()

                @pl.loop(0, chunk // 2)
                def _(rh):
                    for u in range(2):
                        r = rh * 2 + u
                        wvec = plsc.bitcast(
                            plsc.load_gather(
                                eb_v.at[q, 2], [jnp.broadcast_to(r, (L,))]
                            ),
                            jnp.float32,
                        )
                        for j in range(jcount):
                            sl = pl.ds(j * L, L)
                            rv[r, sl] = rv[r, sl] * wvec

                pltpu.async_copy(rv, acc_sh.at[eb_v.at[q, 1]], ssem[b], add=True)

                @pl.when(kk + 1 < nch)
                def _():
                    @pl.when(kk >= 1)
                    def _():
                        pltpu.make_async_copy(
                            rvs[nb], acc_sh.at[eb_v.at[(q + 3) % 4, 1]], ssem[nb]
                        ).wait()

                    @pl.when(kk + 3 < nch)
                    def _():
                        pltpu.async_copy(
                            e_hbm.at[c, s, kk + 3],
                            eb_v.at[(q + 3) % 4],
                            esem[(q + 3) % 4],
                        )

                    pltpu.make_async_copy(
                        e_hbm.at[c, s, kk + 1], eb_v.at[(q + 1) % 4],
                        esem[(q + 1) % 4],
                    ).wait()
                    pltpu.async_copy(
                        h_hbm.at[eb_v.at[(q + 1) % 4, 0]], rvs[nb], gsem[nb]
                    )

        pltpu.make_async_copy(
            rvs[(nch - 2) % 2], acc_sh.at[eb_v.at[(nch - 2) % 4, 1]],
            ssem[(nch - 2) % 2],
        ).wait()
        pltpu.make_async_copy(
            rvs[(nch - 1) % 2], acc_sh.at[eb_v.at[(nch - 1) % 4, 1]],
            ssem[(nch - 1) % 2],
        ).wait()
        plsc.subcore_barrier()

        @pl.loop(0, nzchunks)
        def _(i):
            pltpu.sync_copy(
                acc_sh.at[pl.ds(base + i * 16, 16)],
                s_hbm.at[pl.ds(c * N + base + i * 16, 16)],
            )

    return k(h2d, edata)


def kernel(x, edge_index, W1, a_s1, a_d1, b1, W2, a_s2, a_d2, b2, Wout, bout):
    src = edge_index[0]
    dst = edge_index[1]
    srcA = src.reshape(NC * NS, EA)
    dstA = dst.reshape(NC * NS, EA)

    c1, n1 = 100, 200
    srcp1 = jnp.stack([src, src + N]).reshape(2, NS, n1, c1)
    dstp1 = jnp.stack([dst, dst]).reshape(2, NS, n1, c1)

    c2, n2 = 100, 100
    srcp2 = src.reshape(2, NS, n2, c2)
    dstp2 = dst.reshape(2, NS, n2, c2)

    h3, as1v, ad1v = _tc_embed(x, W1, a_s1, a_d1)
    w1 = _sc_edge_w(as1v, ad1v, srcA, dstA)
    w1b = lax.bitcast_convert_type(w1.reshape(E), jnp.int32)
    wp1 = jnp.stack([w1b, w1b]).reshape(2, NS, n1, 1, c1)
    edata1 = jnp.concatenate(
        [srcp1.reshape(2, NS, n1, 1, c1), dstp1.reshape(2, NS, n1, 1, c1), wp1],
        axis=3,
    )
    S1 = _sc_scatter(h3.reshape(2 * N, HB), edata1, c1, n1)

    h2, as2v, ad2v = _tc_norm_embed(S1.reshape(2, N, HB), W2, b1,
                                    a_s2, a_d2, 200)
    w2 = _sc_edge_w(as2v, ad2v, srcA, dstA)
    wp2 = lax.bitcast_convert_type(w2, jnp.int32).reshape(2, NS, n2, 1, c2)
    edata2 = jnp.concatenate(
        [srcp2.reshape(2, NS, n2, 1, c2), dstp2.reshape(2, NS, n2, 1, c2), wp2],
        axis=3,
    )
    S2 = _sc_scatter(h2, edata2, c2, n2)

    return _tc_final(S2.reshape(2, N, HB), Wout, b2, bout, 100)


# speedup vs baseline: 26.8691x; 1.1565x over previous
import functools

import jax
import jax.numpy as jnp
from jax import lax
from jax.experimental import pallas as pl
from jax.experimental.pallas import tpu as pltpu
from jax.experimental.pallas import tpu_sc as plsc

N = 10000
E = 320000
NC = 2
NS = 16
L = 16
HB = 128
EA = E // (NC * NS)


def _tc_embed(x, W, att_s, att_d):
    n, h_true = x.shape[0], W.shape[1]

    def body(x_ref, w_ref, avs_ref, avd_ref, h_ref, as_ref, ad_ref):
        h = jnp.dot(x_ref[...], w_ref[...], preferred_element_type=jnp.float32)
        as_ref[...] = jnp.sum(h * avs_ref[...][None, :], axis=1)
        ad_ref[...] = jnp.sum(h * avd_ref[...][None, :], axis=1)
        ones = jnp.ones((n, 1), jnp.float32)
        pad = jnp.zeros((n, 2 * HB - h_true - 1), jnp.float32)
        hp = jnp.concatenate([h, ones, pad], axis=1)
        h_ref[0] = hp[:, :HB]
        h_ref[1] = hp[:, HB:]

    return pl.pallas_call(
        body,
        out_shape=(
            jax.ShapeDtypeStruct((2, n, HB), jnp.float32),
            jax.ShapeDtypeStruct((n,), jnp.float32),
            jax.ShapeDtypeStruct((n,), jnp.float32),
        ),
    )(x, W, att_s, att_d)


def _tc_norm_embed(S3, W, b, att_s, att_d, h_true_in):
    n = S3.shape[1]
    h_true = W.shape[1]

    def body(s_ref, w_ref, b_ref, avs_ref, avd_ref, h_ref, as_ref, ad_ref):
        sfull = jnp.concatenate([s_ref[0], s_ref[1]], axis=1)
        S = sfull[:, :h_true_in]
        D = sfull[:, h_true_in]
        xin = jax.nn.relu(S / (D[:, None] + 1e-16) + b_ref[...][None, :])
        h = jnp.dot(xin, w_ref[...], preferred_element_type=jnp.float32)
        as_ref[...] = jnp.sum(h * avs_ref[...][None, :], axis=1)
        ad_ref[...] = jnp.sum(h * avd_ref[...][None, :], axis=1)
        ones = jnp.ones((n, 1), jnp.float32)
        pad = jnp.zeros((n, HB - h_true - 1), jnp.float32)
        h_ref[...] = jnp.concatenate([h, ones, pad], axis=1)

    return pl.pallas_call(
        body,
        out_shape=(
            jax.ShapeDtypeStruct((n, HB), jnp.float32),
            jax.ShapeDtypeStruct((n,), jnp.float32),
            jax.ShapeDtypeStruct((n,), jnp.float32),
        ),
    )(S3, W, b, att_s, att_d)


def _tc_final(S3, W, b, bout, h_true_in):
    n = S3.shape[1]

    def body(s_ref, w_ref, b_ref, bo_ref, o_ref):
        sfull = s_ref[0] + s_ref[1]
        S = sfull[:, :h_true_in]
        D = sfull[:, h_true_in]
        xin = jax.nn.relu(S / (D[:, None] + 1e-16) + b_ref[...][None, :])
        o_ref[...] = (
            jnp.dot(xin, w_ref[...], preferred_element_type=jnp.float32)
            + bo_ref[...][None, :]
        )

    return pl.pallas_call(
        body,
        out_shape=jax.ShapeDtypeStruct((n, W.shape[1]), jnp.float32),
    )(S3, W, b, bout)


def _sc_edge_w(a_s, a_d, srcA, dstA):
    mesh = plsc.VectorSubcoreMesh(core_axis_name="c", subcore_axis_name="s")

    @functools.partial(
        pl.kernel,
        mesh=mesh,
        compiler_params=pltpu.CompilerParams(needs_layout_passes=False),
        out_type=jax.ShapeDtypeStruct((NC * NS, EA), jnp.float32),
        scratch_types=[
            pltpu.VMEM((N,), jnp.float32),
            pltpu.VMEM((N,), jnp.float32),
            pltpu.VMEM((EA,), jnp.int32),
            pltpu.VMEM((EA,), jnp.int32),
            pltpu.VMEM((EA,), jnp.float32),
        ],
    )
    def k(as_hbm, ad_hbm, src_hbm, dst_hbm, w_hbm, as_v, ad_v, src_v, dst_v, w_v):
        wid = lax.axis_index("c") * NS + lax.axis_index("s")
        pltpu.sync_copy(as_hbm, as_v)
        pltpu.sync_copy(ad_hbm, ad_v)
        pltpu.sync_copy(src_hbm.at[wid], src_v)
        pltpu.sync_copy(dst_hbm.at[wid], dst_v)

        @pl.loop(0, EA // L)
        def _(i):
            sl = pl.ds(i * L, L)
            e = plsc.load_gather(as_v, [src_v[sl]]) + plsc.load_gather(
                ad_v, [dst_v[sl]]
            )
            e = jnp.where(e > 0.0, e, 0.2 * e)
            w_v[sl] = jnp.exp(e)

        pltpu.sync_copy(w_v, w_hbm.at[wid])

    return k(a_s, a_d, srcA, dstA)


def _sc_scatter(h2d, edata, chunk, nch):
    mesh = plsc.VectorSubcoreMesh(core_axis_name="c", subcore_axis_name="s")
    jcount = HB // L

    @functools.partial(
        pl.kernel,
        mesh=mesh,
        compiler_params=pltpu.CompilerParams(
            needs_layout_passes=False, use_tc_tiling_on_sc=False
        ),
        out_type=jax.ShapeDtypeStruct((2 * N, HB), jnp.float32),
        scratch_types=[
            pltpu.VMEM_SHARED((N, HB), jnp.float32),
            pltpu.VMEM((4, 3, chunk), jnp.int32),
            pltpu.VMEM((chunk, HB), jnp.float32),
            pltpu.VMEM((chunk, HB), jnp.float32),
            pltpu.SemaphoreType.DMA,
            pltpu.SemaphoreType.DMA,
            pltpu.SemaphoreType.DMA,
            pltpu.SemaphoreType.DMA,
            pltpu.SemaphoreType.DMA,
            pltpu.SemaphoreType.DMA,
            pltpu.SemaphoreType.DMA,
            pltpu.SemaphoreType.DMA,
        ],
    )
    def k(h_hbm, e_hbm, s_hbm, acc_sh, eb_v, rv0, rv1,
          g0, g1, s0, s1, e0, e1, e2, e3):
        c = lax.axis_index("c")
        s = lax.axis_index("s")
        rvs = (rv0, rv1)
        gsem = (g0, g1)
        ssem = (s0, s1)
        esem = (e0, e1, e2, e3)

        @pl.loop(0, 16)
        def _(r):
            for j in range(jcount):
                rv0[r, pl.ds(j * L, L)] = jnp.zeros((L,), jnp.float32)

        base = s * 624
        nzchunks = jnp.where(s == NS - 1, 40, 39)

        @pl.loop(0, nzchunks)
        def _(i):
            pltpu.sync_copy(
                rv0.at[pl.ds(0, 16)], acc_sh.at[pl.ds(base + i * 16, 16)]
            )

        plsc.subcore_barrier()

        for t in range(3):
            pltpu.async_copy(e_hbm.at[c, s, t], eb_v.at[t], esem[t])
        pltpu.make_async_copy(e_hbm.at[c, s, 0], eb_v.at[0], esem[0]).wait()
        pltpu.async_copy(h_hbm.at[eb_v.at[0, 0]], rv0, g0)

        @pl.loop(0, nch // 4)
        def _(g):
            for q in range(4):
                b = q % 2
                nb = 1 - b
                kk = g * 4 + q
                rv = rvs[b]
                pltpu.make_async_copy(h_hbm.at[eb_v.at[q, 0]], rv, gsem[b]).wait()

                @plsc.parallel_loop(0, chunk, unroll=4)
                def _(r):
                    wvec = plsc.bitcast(
                        plsc.load_gather(
                            eb_v.at[q, 2], [jnp.broadcast_to(r, (L,))]
                        ),
                        jnp.float32,
                    )
                    for j in range(jcount):
                        sl = pl.ds(j * L, L)
                        rv[r, sl] = rv[r, sl] * wvec

                pltpu.async_copy(rv, acc_sh.at[eb_v.at[q, 1]], ssem[b], add=True)

                @pl.when(kk + 1 < nch)
                def _():
                    @pl.when(kk >= 1)
                    def _():
                        pltpu.make_async_copy(
                            rvs[nb], acc_sh.at[eb_v.at[(q + 3) % 4, 1]], ssem[nb]
                        ).wait()

                    @pl.when(kk + 3 < nch)
                    def _():
                        pltpu.async_copy(
                            e_hbm.at[c, s, kk + 3],
                            eb_v.at[(q + 3) % 4],
                            esem[(q + 3) % 4],
                        )

                    pltpu.make_async_copy(
                        e_hbm.at[c, s, kk + 1], eb_v.at[(q + 1) % 4],
                        esem[(q + 1) % 4],
                    ).wait()
                    pltpu.async_copy(
                        h_hbm.at[eb_v.at[(q + 1) % 4, 0]], rvs[nb], gsem[nb]
                    )

        pltpu.make_async_copy(
            rvs[(nch - 2) % 2], acc_sh.at[eb_v.at[(nch - 2) % 4, 1]],
            ssem[(nch - 2) % 2],
        ).wait()
        pltpu.make_async_copy(
            rvs[(nch - 1) % 2], acc_sh.at[eb_v.at[(nch - 1) % 4, 1]],
            ssem[(nch - 1) % 2],
        ).wait()
        plsc.subcore_barrier()

        @pl.loop(0, nzchunks)
        def _(i):
            pltpu.sync_copy(
                acc_sh.at[pl.ds(base + i * 16, 16)],
                s_hbm.at[pl.ds(c * N + base + i * 16, 16)],
            )

    return k(h2d, edata)


def kernel(x, edge_index, W1, a_s1, a_d1, b1, W2, a_s2, a_d2, b2, Wout, bout):
    src = edge_index[0]
    dst = edge_index[1]
    srcA = src.reshape(NC * NS, EA)
    dstA = dst.reshape(NC * NS, EA)

    c1, n1 = 100, 200
    srcp1 = jnp.stack([src, src + N]).reshape(2, NS, n1, c1)
    dstp1 = jnp.stack([dst, dst]).reshape(2, NS, n1, c1)

    c2, n2 = 100, 100
    srcp2 = src.reshape(2, NS, n2, c2)
    dstp2 = dst.reshape(2, NS, n2, c2)

    h3, as1v, ad1v = _tc_embed(x, W1, a_s1, a_d1)
    w1 = _sc_edge_w(as1v, ad1v, srcA, dstA)
    w1b = lax.bitcast_convert_type(w1.reshape(E), jnp.int32)
    wp1 = jnp.stack([w1b, w1b]).reshape(2, NS, n1, 1, c1)
    edata1 = jnp.concatenate(
        [srcp1.reshape(2, NS, n1, 1, c1), dstp1.reshape(2, NS, n1, 1, c1), wp1],
        axis=3,
    )
    S1 = _sc_scatter(h3.reshape(2 * N, HB), edata1, c1, n1)

    h2, as2v, ad2v = _tc_norm_embed(S1.reshape(2, N, HB), W2, b1,
                                    a_s2, a_d2, 200)
    w2 = _sc_edge_w(as2v, ad2v, srcA, dstA)
    wp2 = lax.bitcast_convert_type(w2, jnp.int32).reshape(2, NS, n2, 1, c2)
    edata2 = jnp.concatenate(
        [srcp2.reshape(2, NS, n2, 1, c2), dstp2.reshape(2, NS, n2, 1, c2), wp2],
        axis=3,
    )
    S2 = _sc_scatter(h2, edata2, c2, n2)

    return _tc_final(S2.reshape(2, N, HB), Wout, b2, bout, 100)
